# baseline (speedup 1.0000x reference)
import functools

import jax
import jax.numpy as jnp
from jax import lax
from jax.experimental import pallas as pl
from jax.experimental.pallas import tpu as pltpu
from jax.experimental.pallas import tpu_sc as plsc

VOCAB = 1000000
EMBED_DIM = 64
BATCH = 4096
SEQ = 200

_INFO = plsc.get_sparse_core_info()
_NC, _NS = _INFO.num_cores, _INFO.num_subcores
_NW = _NC * _NS
_ROWS = SEQ * BATCH
_IW = 128
_NIDX = _ROWS // _IW
_IDX_PER_W = _NIDX // _NW
_GRP = 8
_STEPS = _IDX_PER_W // _GRP


def _sc_gather(table, idx2d):
    mesh = plsc.VectorSubcoreMesh(core_axis_name="c", subcore_axis_name="s")

    @functools.partial(
        pl.kernel,
        mesh=mesh,
        out_type=jax.ShapeDtypeStruct((_ROWS, EMBED_DIM), jnp.float32),
        scratch_types=[
            pltpu.VMEM((_IDX_PER_W, _IW), jnp.int32),
            pltpu.VMEM((_GRP * _IW, EMBED_DIM), jnp.float32),
            pltpu.SemaphoreType.DMA,
        ],
        compiler_params=pltpu.CompilerParams(use_tc_tiling_on_sc=False),
    )
    def k(table_hbm, idx_hbm, out_hbm, idx_v, rows_v, sem):
        wid = lax.axis_index("s") * _NC + lax.axis_index("c")
        idx_base = wid * _IDX_PER_W
        pltpu.sync_copy(idx_hbm.at[pl.ds(idx_base, _IDX_PER_W), :], idx_v)

        def step(g, carry):
            copies = []
            for j in range(_GRP):
                copies.append(pltpu.async_copy(
                    table_hbm.at[idx_v.at[g * _GRP + j]],
                    rows_v.at[pl.ds(j * _IW, _IW), :],
                    sem,
                ))
            for c in copies:
                c.wait()
            out_row = (idx_base + g * _GRP) * _IW
            pltpu.sync_copy(
                rows_v, out_hbm.at[pl.ds(out_row, _GRP * _IW), :])
            return carry

        lax.fori_loop(0, _STEPS, step, 0)

    return k(table, idx2d)


def kernel(x, table):
    idx2d = jnp.transpose(x).reshape(_NIDX, _IW)
    out = _sc_gather(table, idx2d)
    return out.reshape(SEQ, BATCH, EMBED_DIM)


# speedup vs baseline: 1.0150x; 1.0150x over previous
import functools

import jax
import jax.numpy as jnp
from jax import lax
from jax.experimental import pallas as pl
from jax.experimental.pallas import tpu as pltpu
from jax.experimental.pallas import tpu_sc as plsc

VOCAB = 1000000
EMBED_DIM = 64
BATCH = 4096
SEQ = 200

_INFO = plsc.get_sparse_core_info()
_NC, _NS = _INFO.num_cores, _INFO.num_subcores
_NW = _NC * _NS
_ROWS = SEQ * BATCH
_IW = 128
_NIDX = _ROWS // _IW
_JPW = _NIDX // _NW
_S = 8
_LA = 4
_NBLK = _JPW // _S


def _sc_gather(table, idx2d):
    mesh = plsc.VectorSubcoreMesh(core_axis_name="c", subcore_axis_name="s")

    @functools.partial(
        pl.kernel,
        mesh=mesh,
        out_type=jax.ShapeDtypeStruct((_ROWS, EMBED_DIM), jnp.float32),
        scratch_types=[
            pltpu.VMEM((_JPW, _IW), jnp.int32),
            pltpu.VMEM((_S, _IW, EMBED_DIM), jnp.float32),
            pltpu.SemaphoreType.DMA((_S,)),
            pltpu.SemaphoreType.DMA((_S,)),
        ],
        compiler_params=pltpu.CompilerParams(use_tc_tiling_on_sc=False),
    )
    def k(table_hbm, idx_hbm, out_hbm, idx_v, rows_v, sem_g, sem_w):
        wid = lax.axis_index("s") * _NC + lax.axis_index("c")
        idx_base = wid * _JPW
        pltpu.sync_copy(idx_hbm.at[pl.ds(idx_base, _JPW), :], idx_v)

        def fire_g(j, slot):
            return pltpu.async_copy(
                table_hbm.at[idx_v.at[j]], rows_v.at[slot], sem_g.at[slot])

        def fire_w(j, slot):
            return pltpu.async_copy(
                rows_v.at[slot],
                out_hbm.at[pl.ds((idx_base + j) * _IW, _IW), :],
                sem_w.at[slot])

        def wait_g(slot):
            pltpu.make_async_copy(
                table_hbm.at[idx_v.at[0]], rows_v.at[slot],
                sem_g.at[slot]).wait()

        def wait_w(slot):
            pltpu.make_async_copy(
                rows_v.at[slot],
                out_hbm.at[pl.ds(0, _IW), :], sem_w.at[slot]).wait()

        for j in range(_LA):
            fire_g(j, j % _S)

        for u in range(_S):
            j = u
            if j + _LA < _JPW:
                if j - _LA >= 0:
                    wait_w((j + _LA) % _S)
                fire_g(j + _LA, (j + _LA) % _S)
            wait_g(u)
            fire_w(j, u)

        def block(blk, carry):
            for u in range(_S):
                j = blk * _S + u
                wait_w((u + _LA) % _S)
                fire_g(j + _LA, (u + _LA) % _S)
                wait_g(u)
                fire_w(j, u)
            return carry

        lax.fori_loop(1, _NBLK - 1, block, 0)

        for u in range(_S):
            j = (_NBLK - 1) * _S + u
            wait_w((u + _LA) % _S)
            if j + _LA < _JPW:
                fire_g(j + _LA, (u + _LA) % _S)
            wait_g(u)
            fire_w(j, u)

        for u in range(_S - _LA, _S):
            wait_w(u)

    return k(table, idx2d)


def kernel(x, table):
    idx2d = jnp.transpose(x).reshape(_NIDX, _IW)
    out = _sc_gather(table, idx2d)
    return out.reshape(SEQ, BATCH, EMBED_DIM)


# speedup vs baseline: 1.3571x; 1.3371x over previous
import functools

import jax
import jax.numpy as jnp
from jax import lax
from jax.experimental import pallas as pl
from jax.experimental.pallas import tpu as pltpu
from jax.experimental.pallas import tpu_sc as plsc

VOCAB = 1000000
EMBED_DIM = 64
BATCH = 4096
SEQ = 200

_INFO = plsc.get_sparse_core_info()
_NC, _NS = _INFO.num_cores, _INFO.num_subcores
_NW = _NC * _NS
_ROWS = SEQ * BATCH
_IW = 128
_NIDX = _ROWS // _IW
_JPW = _NIDX // _NW
_S = 8
_LA = 4
_NBLK = _JPW // _S


def _sc_gather(table, idx2d):
    mesh = plsc.VectorSubcoreMesh(core_axis_name="c", subcore_axis_name="s")

    @functools.partial(
        pl.kernel,
        mesh=mesh,
        out_type=jax.ShapeDtypeStruct((_ROWS, 2 * EMBED_DIM), jnp.float32),
        scratch_types=[
            pltpu.VMEM((_JPW, _IW), jnp.int32),
            pltpu.VMEM((_S, _IW, EMBED_DIM), jnp.float32),
            pltpu.SemaphoreType.DMA((_S,)),
            pltpu.SemaphoreType.DMA((_S,)),
        ],
        compiler_params=pltpu.CompilerParams(use_tc_tiling_on_sc=False),
    )
    def k(table_hbm, idx_hbm, out_hbm, idx_v, rows_v, sem_g, sem_w):
        wid = lax.axis_index("s") * _NC + lax.axis_index("c")
        idx_base = wid * _JPW
        pltpu.sync_copy(idx_hbm.at[pl.ds(idx_base, _JPW), :], idx_v)

        def fire_g(j, slot):
            return pltpu.async_copy(
                table_hbm.at[idx_v.at[j]], rows_v.at[slot], sem_g.at[slot])

        def fire_w(j, slot):
            return pltpu.async_copy(
                rows_v.at[slot],
                out_hbm.at[pl.ds((idx_base + j) * _IW, _IW),
                           pl.ds(0, EMBED_DIM)],
                sem_w.at[slot])

        def wait_g(slot):
            pltpu.make_async_copy(
                table_hbm.at[idx_v.at[0]], rows_v.at[slot],
                sem_g.at[slot]).wait()

        def wait_w(slot):
            pltpu.make_async_copy(
                rows_v.at[slot],
                out_hbm.at[pl.ds(0, _IW), pl.ds(0, EMBED_DIM)],
                sem_w.at[slot]).wait()

        for j in range(_LA):
            fire_g(j, j % _S)

        for u in range(_S):
            j = u
            if j + _LA < _JPW:
                if j - _LA >= 0:
                    wait_w((j + _LA) % _S)
                fire_g(j + _LA, (j + _LA) % _S)
            wait_g(u)
            fire_w(j, u)

        def block(blk, carry):
            for u in range(_S):
                j = blk * _S + u
                wait_w((u + _LA) % _S)
                fire_g(j + _LA, (u + _LA) % _S)
                wait_g(u)
                fire_w(j, u)
            return carry

        lax.fori_loop(1, _NBLK - 1, block, 0)

        for u in range(_S):
            j = (_NBLK - 1) * _S + u
            wait_w((u + _LA) % _S)
            if j + _LA < _JPW:
                fire_g(j + _LA, (u + _LA) % _S)
            wait_g(u)
            fire_w(j, u)

        for u in range(_S - _LA, _S):
            wait_w(u)

    return k(table, idx2d)


def kernel(x, table):
    idx2d = jnp.transpose(x).reshape(_NIDX, _IW)
    out = _sc_gather(table, idx2d)
    return out[:, :EMBED_DIM].reshape(SEQ, BATCH, EMBED_DIM)
